# Initial kernel scaffold; baseline (speedup 1.0000x reference)
import jax
import jax.numpy as jnp
from jax.experimental import pallas as pl


def kernel(user_table, movie_table, user_id, movie_id):
    raise NotImplementedError("write your pallas kernel here")


# speedup vs baseline: 3.1346x; 3.1346x over previous
import jax
import jax.numpy as jnp
from jax import lax
from jax.experimental import pallas as pl
from jax.experimental.pallas import tpu as pltpu

_EPS = 1e-8
_TILE_B = 256


def _cosine_gather_kernel(uid_ref, mid_ref, ut_ref, mt_ref, o_ref, u_t, m_t):
    tb = o_ref.shape[2]
    s = tb + 1
    for mi in range(tb):
        iu = pl.multiple_of(uid_ref[0, 0, mi], 2)
        im = pl.multiple_of(mid_ref[0, 0, mi], 2)
        u_t[mi:mi + 2 * s:s, :] = ut_ref[pl.ds(iu, 2), :]
        m_t[mi:mi + 2 * s:s, :] = mt_ref[pl.ds(im, 2), :]
    u0 = u_t[pl.ds(0, tb), :]
    u1 = u_t[pl.ds(s, tb), :]
    m0 = m_t[pl.ds(0, tb), :]
    m1 = m_t[pl.ds(s, tb), :]
    pum = u0 * m0 + u1 * m1
    puu = u0 * u0 + u1 * u1
    pmm = m0 * m0 + m1 * m1
    cat = jnp.concatenate([pum, puu, pmm], axis=0)
    ones = jnp.ones((1, 128), jnp.float32)
    sums = lax.dot_general(ones, cat, (((1,), (1,)), ((), ())),
                           preferred_element_type=jnp.float32)
    dum = sums[:, :tb]
    nu = jnp.sqrt(sums[:, tb:2 * tb])
    nm = jnp.sqrt(sums[:, 2 * tb:])
    o_ref[0] = dum / jnp.maximum(nu * nm, _EPS)


def kernel(user_table, movie_table, user_id, movie_id):
    B = int(user_id.shape[0])
    Nu, D = user_table.shape
    Nm = movie_table.shape[0]

    tile_b = min(_TILE_B, max(128, B))
    num_tiles = pl.cdiv(B, tile_b)
    Bp = num_tiles * tile_b

    chunks = D // 128
    uid = user_id.astype(jnp.int32) * chunks
    mid = movie_id.astype(jnp.int32) * chunks
    pad = Bp - B
    if pad:
        uid = jnp.concatenate([uid, jnp.zeros((pad,), jnp.int32)])
        mid = jnp.concatenate([mid, jnp.zeros((pad,), jnp.int32)])
    uid3 = uid.reshape(num_tiles, 1, tile_b)
    mid3 = mid.reshape(num_tiles, 1, tile_b)

    ut2 = user_table.reshape(Nu * chunks, 128)
    mt2 = movie_table.reshape(Nm * chunks, 128)

    tables_bytes = (user_table.size + movie_table.size) * 4
    cost = pl.CostEstimate(
        flops=int(8 * Bp * D),
        transcendentals=int(2 * Bp),
        bytes_accessed=int(tables_bytes + 3 * Bp * 4),
    )
    vmem_limit = int(tables_bytes + 6 * (tile_b + 1) * 128 * 4 + (8 << 20))

    out = pl.pallas_call(
        _cosine_gather_kernel,
        out_shape=jax.ShapeDtypeStruct((num_tiles, 1, tile_b), jnp.float32),
        grid=(num_tiles,),
        in_specs=[
            pl.BlockSpec((1, 1, tile_b), lambda i: (i, 0, 0),
                         memory_space=pltpu.SMEM),
            pl.BlockSpec((1, 1, tile_b), lambda i: (i, 0, 0),
                         memory_space=pltpu.SMEM),
            pl.BlockSpec((Nu * chunks, 128), lambda i: (0, 0)),
            pl.BlockSpec((Nm * chunks, 128), lambda i: (0, 0)),
        ],
        out_specs=pl.BlockSpec((1, 1, tile_b), lambda i: (i, 0, 0)),
        scratch_shapes=[
            pltpu.VMEM((2 * (tile_b + 1), 128), jnp.float32),
            pltpu.VMEM((2 * (tile_b + 1), 128), jnp.float32),
        ],
        compiler_params=pltpu.CompilerParams(
            dimension_semantics=("arbitrary",),
            vmem_limit_bytes=vmem_limit,
        ),
        cost_estimate=cost,
    )(uid3, mid3, ut2, mt2)
    return out.reshape(-1)[:B]


# speedup vs baseline: 4.0832x; 1.3026x over previous
import jax
import jax.numpy as jnp
from jax import lax
from jax.experimental import pallas as pl
from jax.experimental.pallas import tpu as pltpu

_EPS = 1e-8
_TILE_B = 2048


def _cosine_gather_kernel(uid_ref, mid_ref, ut_ref, mt_ref, o_ref, u_t, m_t):
    tb = o_ref.shape[2]
    s = tb + 1
    for mi in range(tb):
        iu = pl.multiple_of(uid_ref[0, 0, mi], 2)
        im = pl.multiple_of(mid_ref[0, 0, mi], 2)
        u_t[mi:mi + 2 * s:s, :] = ut_ref[pl.ds(iu, 2), :]
        m_t[mi:mi + 2 * s:s, :] = mt_ref[pl.ds(im, 2), :]
    u0 = u_t[pl.ds(0, tb), :]
    u1 = u_t[pl.ds(s, tb), :]
    m0 = m_t[pl.ds(0, tb), :]
    m1 = m_t[pl.ds(s, tb), :]
    pum = u0 * m0 + u1 * m1
    puu = u0 * u0 + u1 * u1
    pmm = m0 * m0 + m1 * m1
    cat = jnp.concatenate([pum, puu, pmm], axis=0)
    ones = jnp.ones((1, 128), jnp.float32)
    sums = lax.dot_general(ones, cat, (((1,), (1,)), ((), ())),
                           preferred_element_type=jnp.float32)
    dum = sums[:, :tb]
    nu = jnp.sqrt(sums[:, tb:2 * tb])
    nm = jnp.sqrt(sums[:, 2 * tb:])
    o_ref[0] = dum / jnp.maximum(nu * nm, _EPS)


def kernel(user_table, movie_table, user_id, movie_id):
    B = int(user_id.shape[0])
    Nu, D = user_table.shape
    Nm = movie_table.shape[0]

    tile_b = min(_TILE_B, max(128, B))
    num_tiles = pl.cdiv(B, tile_b)
    Bp = num_tiles * tile_b

    chunks = D // 128
    uid = user_id.astype(jnp.int32) * chunks
    mid = movie_id.astype(jnp.int32) * chunks
    pad = Bp - B
    if pad:
        uid = jnp.concatenate([uid, jnp.zeros((pad,), jnp.int32)])
        mid = jnp.concatenate([mid, jnp.zeros((pad,), jnp.int32)])
    uid3 = uid.reshape(num_tiles, 1, tile_b)
    mid3 = mid.reshape(num_tiles, 1, tile_b)

    ut2 = user_table.reshape(Nu * chunks, 128)
    mt2 = movie_table.reshape(Nm * chunks, 128)

    tables_bytes = (user_table.size + movie_table.size) * 4
    cost = pl.CostEstimate(
        flops=int(8 * Bp * D),
        transcendentals=int(2 * Bp),
        bytes_accessed=int(tables_bytes + 3 * Bp * 4),
    )
    vmem_limit = int(tables_bytes + 6 * (tile_b + 1) * 128 * 4 + (8 << 20))

    out = pl.pallas_call(
        _cosine_gather_kernel,
        out_shape=jax.ShapeDtypeStruct((num_tiles, 1, tile_b), jnp.float32),
        grid=(num_tiles,),
        in_specs=[
            pl.BlockSpec((1, 1, tile_b), lambda i: (i, 0, 0),
                         memory_space=pltpu.SMEM),
            pl.BlockSpec((1, 1, tile_b), lambda i: (i, 0, 0),
                         memory_space=pltpu.SMEM),
            pl.BlockSpec((Nu * chunks, 128), lambda i: (0, 0)),
            pl.BlockSpec((Nm * chunks, 128), lambda i: (0, 0)),
        ],
        out_specs=pl.BlockSpec((1, 1, tile_b), lambda i: (i, 0, 0)),
        scratch_shapes=[
            pltpu.VMEM((2 * (tile_b + 1), 128), jnp.float32),
            pltpu.VMEM((2 * (tile_b + 1), 128), jnp.float32),
        ],
        compiler_params=pltpu.CompilerParams(
            dimension_semantics=("arbitrary",),
            vmem_limit_bytes=vmem_limit,
        ),
        cost_estimate=cost,
    )(uid3, mid3, ut2, mt2)
    return out.reshape(-1)[:B]


# speedup vs baseline: 4.1810x; 1.0240x over previous
import jax
import jax.numpy as jnp
from jax import lax
from jax.experimental import pallas as pl
from jax.experimental.pallas import tpu as pltpu

_EPS = 1e-8
_TILE_B = 4096


def _cosine_gather_kernel(uid_ref, mid_ref, ut_ref, mt_ref, o_ref, u_t, m_t):
    tb = o_ref.shape[2]
    s = tb + 1
    for mi in range(tb):
        iu = pl.multiple_of(uid_ref[0, 0, mi], 2)
        im = pl.multiple_of(mid_ref[0, 0, mi], 2)
        u_t[mi:mi + 2 * s:s, :] = ut_ref[pl.ds(iu, 2), :]
        m_t[mi:mi + 2 * s:s, :] = mt_ref[pl.ds(im, 2), :]
    u0 = u_t[pl.ds(0, tb), :]
    u1 = u_t[pl.ds(s, tb), :]
    m0 = m_t[pl.ds(0, tb), :]
    m1 = m_t[pl.ds(s, tb), :]
    pum = u0 * m0 + u1 * m1
    puu = u0 * u0 + u1 * u1
    pmm = m0 * m0 + m1 * m1
    cat = jnp.concatenate([pum, puu, pmm], axis=0)
    ones = jnp.ones((1, 128), jnp.float32)
    sums = lax.dot_general(ones, cat, (((1,), (1,)), ((), ())),
                           preferred_element_type=jnp.float32)
    dum = sums[:, :tb]
    nu = jnp.sqrt(sums[:, tb:2 * tb])
    nm = jnp.sqrt(sums[:, 2 * tb:])
    o_ref[0] = dum / jnp.maximum(nu * nm, _EPS)


def kernel(user_table, movie_table, user_id, movie_id):
    B = int(user_id.shape[0])
    Nu, D = user_table.shape
    Nm = movie_table.shape[0]

    tile_b = min(_TILE_B, max(128, B))
    num_tiles = pl.cdiv(B, tile_b)
    Bp = num_tiles * tile_b

    chunks = D // 128
    uid = user_id.astype(jnp.int32) * chunks
    mid = movie_id.astype(jnp.int32) * chunks
    pad = Bp - B
    if pad:
        uid = jnp.concatenate([uid, jnp.zeros((pad,), jnp.int32)])
        mid = jnp.concatenate([mid, jnp.zeros((pad,), jnp.int32)])
    uid3 = uid.reshape(num_tiles, 1, tile_b)
    mid3 = mid.reshape(num_tiles, 1, tile_b)

    ut2 = user_table.reshape(Nu * chunks, 128)
    mt2 = movie_table.reshape(Nm * chunks, 128)

    tables_bytes = (user_table.size + movie_table.size) * 4
    cost = pl.CostEstimate(
        flops=int(8 * Bp * D),
        transcendentals=int(2 * Bp),
        bytes_accessed=int(tables_bytes + 3 * Bp * 4),
    )
    vmem_limit = int(tables_bytes + 6 * (tile_b + 1) * 128 * 4 + (8 << 20))

    out = pl.pallas_call(
        _cosine_gather_kernel,
        out_shape=jax.ShapeDtypeStruct((num_tiles, 1, tile_b), jnp.float32),
        grid=(num_tiles,),
        in_specs=[
            pl.BlockSpec((1, 1, tile_b), lambda i: (i, 0, 0),
                         memory_space=pltpu.SMEM),
            pl.BlockSpec((1, 1, tile_b), lambda i: (i, 0, 0),
                         memory_space=pltpu.SMEM),
            pl.BlockSpec((Nu * chunks, 128), lambda i: (0, 0)),
            pl.BlockSpec((Nm * chunks, 128), lambda i: (0, 0)),
        ],
        out_specs=pl.BlockSpec((1, 1, tile_b), lambda i: (i, 0, 0)),
        scratch_shapes=[
            pltpu.VMEM((2 * (tile_b + 1), 128), jnp.float32),
            pltpu.VMEM((2 * (tile_b + 1), 128), jnp.float32),
        ],
        compiler_params=pltpu.CompilerParams(
            dimension_semantics=("arbitrary",),
            vmem_limit_bytes=vmem_limit,
        ),
        cost_estimate=cost,
    )(uid3, mid3, ut2, mt2)
    return out.reshape(-1)[:B]
